# baseline (speedup 1.0000x reference)
import functools

import jax
import jax.numpy as jnp
from jax import lax
from jax.experimental import pallas as pl
from jax.experimental.pallas import tpu as pltpu
from jax.experimental.pallas import tpu_sc as plsc

_NNZ = 268435
_NW = 32
_C = 8384
_NQ = 4
_Q = _C // _NQ
_MAIN = _NW * _C
_TAIL = _NNZ - _MAIN
_TAIL_PAD = 160
_SCALE = float(1.0 / 0.9)
_LANES = 16


def _drop(x, r):
    keep = (r + jnp.float32(0.9)) >= jnp.float32(1.0)
    return jnp.where(keep, x * jnp.float32(_SCALE), jnp.float32(0.0))


def _dropout_body(vals_hbm, rand_hbm, out_hbm,
                  v_v, r_v, o_v, tv_v, tr_v, sem, sem_b, tsem):
    wid = lax.axis_index("s") * 2 + lax.axis_index("c")
    base = wid * _C
    is_tail_worker = wid == 0

    @pl.when(is_tail_worker)
    def _tail_fire():
        pltpu.async_copy(vals_hbm.at[pl.ds(_MAIN, _TAIL)],
                         tv_v.at[pl.ds(0, _TAIL)], tsem)
        pltpu.async_copy(rand_hbm.at[pl.ds(_MAIN, _TAIL)],
                         tr_v.at[pl.ds(0, _TAIL)], tsem)

    hin = []
    for q in range(_NQ):
        hin.append((
            pltpu.async_copy(vals_hbm.at[pl.ds(base + q * _Q, _Q)],
                             v_v.at[pl.ds(q * _Q, _Q)], sem),
            pltpu.async_copy(rand_hbm.at[pl.ds(base + q * _Q, _Q)],
                             r_v.at[pl.ds(q * _Q, _Q)], sem),
        ))

    @pl.when(is_tail_worker)
    def _tail_compute():
        pltpu.make_async_copy(vals_hbm.at[pl.ds(_MAIN, _TAIL)],
                              tv_v.at[pl.ds(0, _TAIL)], tsem).wait()
        pltpu.make_async_copy(rand_hbm.at[pl.ds(_MAIN, _TAIL)],
                              tr_v.at[pl.ds(0, _TAIL)], tsem).wait()
        for j in range(_TAIL_PAD // _LANES):
            o = j * _LANES
            tv_v[pl.ds(o, _LANES)] = _drop(tv_v[pl.ds(o, _LANES)],
                                           tr_v[pl.ds(o, _LANES)])
        pltpu.async_copy(tv_v.at[pl.ds(0, _TAIL)],
                         out_hbm.at[pl.ds(_MAIN, _TAIL)], tsem)

    hout = []
    for q in range(_NQ):
        hin[q][0].wait()
        hin[q][1].wait()

        @plsc.parallel_loop(q * _Q, (q + 1) * _Q, _LANES, unroll=4)
        def _main_q(o):
            o_v[pl.ds(o, _LANES)] = _drop(v_v[pl.ds(o, _LANES)],
                                          r_v[pl.ds(o, _LANES)])

        hout.append(pltpu.async_copy(o_v.at[pl.ds(q * _Q, _Q)],
                                     out_hbm.at[pl.ds(base + q * _Q, _Q)],
                                     sem_b))

    @pl.when(is_tail_worker)
    def _tail_drain():
        pltpu.make_async_copy(tv_v.at[pl.ds(0, _TAIL)],
                              out_hbm.at[pl.ds(_MAIN, _TAIL)], tsem).wait()

    for h in hout:
        h.wait()


_dropout_sc = functools.partial(
    pl.kernel,
    out_type=jax.ShapeDtypeStruct((_NNZ,), jnp.float32),
    mesh=plsc.VectorSubcoreMesh(core_axis_name="c", subcore_axis_name="s"),
    scratch_types=[
        pltpu.VMEM((_C,), jnp.float32),
        pltpu.VMEM((_C,), jnp.float32),
        pltpu.VMEM((_C,), jnp.float32),
        pltpu.VMEM((_TAIL_PAD,), jnp.float32),
        pltpu.VMEM((_TAIL_PAD,), jnp.float32),
        pltpu.SemaphoreType.DMA,
        pltpu.SemaphoreType.DMA,
        pltpu.SemaphoreType.DMA,
    ],
)(_dropout_body)


def kernel(x_indices, x_values, rand_vals):
    out_values = _dropout_sc(x_values, rand_vals)
    return x_indices, out_values


# speedup vs baseline: 1.0358x; 1.0358x over previous
import functools

import jax
import jax.numpy as jnp
from jax import lax
from jax.experimental import pallas as pl
from jax.experimental.pallas import tpu as pltpu
from jax.experimental.pallas import tpu_sc as plsc

_NNZ = 268435
_NC = 1
_NW = _NC * 16
_C = 16768
_NQ = 4
_Q = _C // _NQ
_MAIN = _NW * _C
_TAIL = _NNZ - _MAIN
_TAIL_PAD = 160
_SCALE = float(1.0 / 0.9)
_LANES = 16


def _drop(x, r):
    keep = (r + jnp.float32(0.9)) >= jnp.float32(1.0)
    return jnp.where(keep, x * jnp.float32(_SCALE), jnp.float32(0.0))


def _dropout_body(vals_hbm, rand_hbm, out_hbm,
                  v_v, r_v, o_v, tv_v, tr_v, sem, sem_b, tsem):
    wid = lax.axis_index("s") * _NC + lax.axis_index("c")
    base = wid * _C
    is_tail_worker = wid == 0

    @pl.when(is_tail_worker)
    def _tail_fire():
        pltpu.async_copy(vals_hbm.at[pl.ds(_MAIN, _TAIL)],
                         tv_v.at[pl.ds(0, _TAIL)], tsem)
        pltpu.async_copy(rand_hbm.at[pl.ds(_MAIN, _TAIL)],
                         tr_v.at[pl.ds(0, _TAIL)], tsem)

    hin = []
    for q in range(_NQ):
        hin.append((
            pltpu.async_copy(vals_hbm.at[pl.ds(base + q * _Q, _Q)],
                             v_v.at[pl.ds(q * _Q, _Q)], sem),
            pltpu.async_copy(rand_hbm.at[pl.ds(base + q * _Q, _Q)],
                             r_v.at[pl.ds(q * _Q, _Q)], sem),
        ))

    @pl.when(is_tail_worker)
    def _tail_compute():
        pltpu.make_async_copy(vals_hbm.at[pl.ds(_MAIN, _TAIL)],
                              tv_v.at[pl.ds(0, _TAIL)], tsem).wait()
        pltpu.make_async_copy(rand_hbm.at[pl.ds(_MAIN, _TAIL)],
                              tr_v.at[pl.ds(0, _TAIL)], tsem).wait()
        for j in range(_TAIL_PAD // _LANES):
            o = j * _LANES
            tv_v[pl.ds(o, _LANES)] = _drop(tv_v[pl.ds(o, _LANES)],
                                           tr_v[pl.ds(o, _LANES)])
        pltpu.async_copy(tv_v.at[pl.ds(0, _TAIL)],
                         out_hbm.at[pl.ds(_MAIN, _TAIL)], tsem)

    hout = []
    for q in range(_NQ):
        hin[q][0].wait()
        hin[q][1].wait()

        @plsc.parallel_loop(q * _Q, (q + 1) * _Q, _LANES, unroll=4)
        def _main_q(o):
            o_v[pl.ds(o, _LANES)] = _drop(v_v[pl.ds(o, _LANES)],
                                          r_v[pl.ds(o, _LANES)])

        hout.append(pltpu.async_copy(o_v.at[pl.ds(q * _Q, _Q)],
                                     out_hbm.at[pl.ds(base + q * _Q, _Q)],
                                     sem_b))

    @pl.when(is_tail_worker)
    def _tail_drain():
        pltpu.make_async_copy(tv_v.at[pl.ds(0, _TAIL)],
                              out_hbm.at[pl.ds(_MAIN, _TAIL)], tsem).wait()

    for h in hout:
        h.wait()


_dropout_sc = functools.partial(
    pl.kernel,
    out_type=jax.ShapeDtypeStruct((_NNZ,), jnp.float32),
    mesh=plsc.VectorSubcoreMesh(core_axis_name="c", subcore_axis_name="s",
                                num_cores=_NC),
    scratch_types=[
        pltpu.VMEM((_C,), jnp.float32),
        pltpu.VMEM((_C,), jnp.float32),
        pltpu.VMEM((_C,), jnp.float32),
        pltpu.VMEM((_TAIL_PAD,), jnp.float32),
        pltpu.VMEM((_TAIL_PAD,), jnp.float32),
        pltpu.SemaphoreType.DMA,
        pltpu.SemaphoreType.DMA,
        pltpu.SemaphoreType.DMA,
    ],
)(_dropout_body)


def kernel(x_indices, x_values, rand_vals):
    out_values = _dropout_sc(x_values, rand_vals)
    return x_indices, out_values
